# baseline (speedup 1.0000x reference)
import functools

import jax
import jax.numpy as jnp
from jax import lax
from jax.experimental import pallas as pl
from jax.experimental.pallas import tpu as pltpu
from jax.experimental.pallas import tpu_sc as plsc

LAT = 64
XD = 16
B = 16384
NC, NS, L = 2, 16, 16
NW = NC * NS
RPW = B // NW
CHUNK = 128
NCH = RPW // CHUNK


def _sc_body(ui_hbm, ii_hbm, mf_hbm, ut_hbm, it_hbm, ix1_hbm, ux0_hbm,
             ux1_hbm, out_hbm, uidx, iidx, mfv, ulat, ilat, ix1v, ux0v,
             ux1v, outv, sem):
  wid = lax.axis_index("s") * NC + lax.axis_index("c")

  pltpu.sync_copy(ui_hbm.at[wid], uidx)
  pltpu.sync_copy(ii_hbm.at[wid], iidx)
  pltpu.sync_copy(mf_hbm.at[wid], mfv)

  cps = []
  for j in range(NCH):
    rows = pl.ds(j * CHUNK, CHUNK)
    cps.append(pltpu.async_copy(ut_hbm.at[uidx.at[j]], ulat.at[rows], sem))
    cps.append(pltpu.async_copy(it_hbm.at[iidx.at[j]], ilat.at[rows], sem))
    cps.append(pltpu.async_copy(ix1_hbm.at[iidx.at[j]], ix1v.at[rows], sem))
    cps.append(pltpu.async_copy(ux0_hbm.at[uidx.at[j]], ux0v.at[rows], sem))
    cps.append(pltpu.async_copy(ux1_hbm.at[uidx.at[j]], ux1v.at[rows], sem))
  for cp in cps:
    cp.wait()

  def row_body(r, carry):
    mfr = mfv[r, :]
    u4 = ux1v[r, :] * mfr
    i4 = ux0v[r, :] + (ix1v[r, :] - ux0v[r, :]) * mfr
    ivec = [ilat[r, pl.ds(j * L, L)] for j in range(4)] + [i4]
    uvec = [ulat[r, pl.ds(j * L, L)] for j in range(4)] + [u4]
    mx = jnp.maximum(jnp.maximum(jnp.maximum(ivec[0], ivec[1]),
                                 jnp.maximum(ivec[2], ivec[3])), ivec[4])
    m_s = jnp.max(mx)
    e = [jnp.exp(v - m_s) for v in ivec]
    s = [1.0 / (1.0 + jnp.exp(-v)) for v in uvec]
    evec = (e[0] + e[1]) + (e[2] + e[3]) + e[4]
    pvec = (s[0] * e[0] + s[1] * e[1]) + (s[2] * e[2] + s[3] * e[3]) + s[4] * e[4]
    valvec = jnp.full((L,), jnp.sum(pvec), jnp.float32) / jnp.full(
        (L,), jnp.sum(evec), jnp.float32)
    lane = lax.broadcasted_iota(jnp.int32, (L,), 0)
    plsc.store_scatter(outv, [jnp.full((L,), r, jnp.int32)], valvec,
                       mask=lane == 0)
    return carry

  lax.fori_loop(0, RPW, row_body, 0)
  pltpu.sync_copy(outv, out_hbm.at[wid])


@jax.jit
def kernel(users, items, xij, user_table, item_table, item_xij1_table,
           item_xij0_table, user_xij1_table, user_xij0_table):
  del item_xij0_table
  ui = users.astype(jnp.int32).reshape(NW, NCH, CHUNK)
  ii = items.astype(jnp.int32).reshape(NW, NCH, CHUNK)
  mf = jnp.broadcast_to(xij.astype(jnp.float32)[:, None], (B, XD))
  mf = mf.reshape(NW, RPW, XD)

  mesh = plsc.VectorSubcoreMesh(core_axis_name="c", subcore_axis_name="s")
  run = functools.partial(
      pl.kernel,
      out_type=jax.ShapeDtypeStruct((NW, RPW), jnp.float32),
      mesh=mesh,
      compiler_params=pltpu.CompilerParams(needs_layout_passes=False,
                                           use_tc_tiling_on_sc=False),
      scratch_types=[
          pltpu.VMEM((NCH, CHUNK), jnp.int32),
          pltpu.VMEM((NCH, CHUNK), jnp.int32),
          pltpu.VMEM((RPW, XD), jnp.float32),
          pltpu.VMEM((RPW, LAT), jnp.float32),
          pltpu.VMEM((RPW, LAT), jnp.float32),
          pltpu.VMEM((RPW, XD), jnp.float32),
          pltpu.VMEM((RPW, XD), jnp.float32),
          pltpu.VMEM((RPW, XD), jnp.float32),
          pltpu.VMEM((RPW,), jnp.float32),
          pltpu.SemaphoreType.DMA,
      ],
  )(_sc_body)
  out = run(ui, ii, mf, user_table, item_table, item_xij1_table,
            user_xij0_table, user_xij1_table)
  return out.reshape(B)


# speedup vs baseline: 1.0333x; 1.0333x over previous
import functools

import jax
import jax.numpy as jnp
from jax import lax
from jax.experimental import pallas as pl
from jax.experimental.pallas import tpu as pltpu
from jax.experimental.pallas import tpu_sc as plsc

LAT = 64
XD = 16
B = 16384
NC, NS, L = 2, 16, 16
NW = NC * NS
RPW = B // NW
CHUNK = 128
NCH = RPW // CHUNK


def _sc_body(ui_hbm, ii_hbm, mf_hbm, ut_hbm, it_hbm, ix1_hbm, ux0_hbm,
             ux1_hbm, out_hbm, uidx, iidx, mfv, ulat, ilat, ix1v, ux0v,
             ux1v, outv, sem):
  wid = lax.axis_index("s") * NC + lax.axis_index("c")

  pltpu.sync_copy(ui_hbm.at[wid], uidx)
  pltpu.sync_copy(ii_hbm.at[wid], iidx)
  pltpu.sync_copy(mf_hbm.at[wid], mfv)

  cps = []
  for j in range(NCH):
    rows = pl.ds(j * CHUNK, CHUNK)
    cps.append(pltpu.async_copy(ut_hbm.at[uidx.at[j]], ulat.at[rows], sem))
    cps.append(pltpu.async_copy(it_hbm.at[iidx.at[j]], ilat.at[rows], sem))
    cps.append(pltpu.async_copy(ix1_hbm.at[iidx.at[j]], ix1v.at[rows], sem))
    cps.append(pltpu.async_copy(ux0_hbm.at[uidx.at[j]], ux0v.at[rows], sem))
    cps.append(pltpu.async_copy(ux1_hbm.at[uidx.at[j]], ux1v.at[rows], sem))
  for cp in cps:
    cp.wait()

  @plsc.parallel_loop(0, RPW, 1, unroll=8)
  def row_body(r):
    mfr = mfv[r, :]
    u4 = ux1v[r, :] * mfr
    i4 = ux0v[r, :] + (ix1v[r, :] - ux0v[r, :]) * mfr
    ivec = [ilat[r, pl.ds(j * L, L)] for j in range(4)] + [i4]
    uvec = [ulat[r, pl.ds(j * L, L)] for j in range(4)] + [u4]
    mx = jnp.maximum(jnp.maximum(jnp.maximum(ivec[0], ivec[1]),
                                 jnp.maximum(ivec[2], ivec[3])), ivec[4])
    m_s = jnp.max(mx)
    e = [jnp.exp(v - m_s) for v in ivec]
    s = [1.0 / (1.0 + jnp.exp(-v)) for v in uvec]
    evec = (e[0] + e[1]) + (e[2] + e[3]) + e[4]
    pvec = (s[0] * e[0] + s[1] * e[1]) + (s[2] * e[2] + s[3] * e[3]) + s[4] * e[4]
    valvec = jnp.full((L,), jnp.sum(pvec), jnp.float32) / jnp.full(
        (L,), jnp.sum(evec), jnp.float32)
    lane = lax.broadcasted_iota(jnp.int32, (L,), 0)
    plsc.store_scatter(outv, [jnp.full((L,), r, jnp.int32)], valvec,
                       mask=lane == 0)

  del row_body
  pltpu.sync_copy(outv, out_hbm.at[wid])


@jax.jit
def kernel(users, items, xij, user_table, item_table, item_xij1_table,
           item_xij0_table, user_xij1_table, user_xij0_table):
  del item_xij0_table
  ui = users.astype(jnp.int32).reshape(NW, NCH, CHUNK)
  ii = items.astype(jnp.int32).reshape(NW, NCH, CHUNK)
  mf = jnp.broadcast_to(xij.astype(jnp.float32)[:, None], (B, XD))
  mf = mf.reshape(NW, RPW, XD)

  mesh = plsc.VectorSubcoreMesh(core_axis_name="c", subcore_axis_name="s")
  run = functools.partial(
      pl.kernel,
      out_type=jax.ShapeDtypeStruct((NW, RPW), jnp.float32),
      mesh=mesh,
      compiler_params=pltpu.CompilerParams(needs_layout_passes=False,
                                           use_tc_tiling_on_sc=False),
      scratch_types=[
          pltpu.VMEM((NCH, CHUNK), jnp.int32),
          pltpu.VMEM((NCH, CHUNK), jnp.int32),
          pltpu.VMEM((RPW, XD), jnp.float32),
          pltpu.VMEM((RPW, LAT), jnp.float32),
          pltpu.VMEM((RPW, LAT), jnp.float32),
          pltpu.VMEM((RPW, XD), jnp.float32),
          pltpu.VMEM((RPW, XD), jnp.float32),
          pltpu.VMEM((RPW, XD), jnp.float32),
          pltpu.VMEM((RPW,), jnp.float32),
          pltpu.SemaphoreType.DMA,
      ],
  )(_sc_body)
  out = run(ui, ii, mf, user_table, item_table, item_xij1_table,
            user_xij0_table, user_xij1_table)
  return out.reshape(B)


# speedup vs baseline: 1.0406x; 1.0071x over previous
import functools

import jax
import jax.numpy as jnp
from jax import lax
from jax.experimental import pallas as pl
from jax.experimental.pallas import tpu as pltpu
from jax.experimental.pallas import tpu_sc as plsc

LAT = 64
XD = 16
B = 16384
NC, NS, L = 2, 16, 16
NW = NC * NS
RPW = B // NW
CHUNK = 128
NCH = RPW // CHUNK


def _sc_body(u_hbm, i_hbm, x_hbm, ut_hbm, it_hbm, ix1_hbm, ux0_hbm,
             ux1_hbm, out_hbm, uidx, iidx, xv, ulat, ilat, ix1v, ux0v,
             ux1v, outv, sem):
  wid = lax.axis_index("s") * NC + lax.axis_index("c")
  base = wid * RPW

  pltpu.sync_copy(u_hbm.at[pl.ds(base, RPW)], uidx)
  pltpu.sync_copy(i_hbm.at[pl.ds(base, RPW)], iidx)
  pltpu.sync_copy(x_hbm.at[pl.ds(base, RPW)], xv)

  cps = []
  for j in range(NCH):
    rows = pl.ds(j * CHUNK, CHUNK)
    cps.append(pltpu.async_copy(ut_hbm.at[uidx.at[rows]], ulat.at[rows], sem))
    cps.append(pltpu.async_copy(it_hbm.at[iidx.at[rows]], ilat.at[rows], sem))
    cps.append(pltpu.async_copy(ix1_hbm.at[iidx.at[rows]], ix1v.at[rows], sem))
    cps.append(pltpu.async_copy(ux0_hbm.at[uidx.at[rows]], ux0v.at[rows], sem))
    cps.append(pltpu.async_copy(ux1_hbm.at[uidx.at[rows]], ux1v.at[rows], sem))
  for cp in cps:
    cp.wait()

  lane = lax.broadcasted_iota(jnp.int32, (L,), 0)

  @plsc.parallel_loop(0, RPW, 1, unroll=8)
  def row_body(r):
    rsplat = jnp.full((L,), r, jnp.int32)
    mfr = plsc.load_gather(xv, [rsplat]).astype(jnp.float32)
    u4 = ux1v[r, :] * mfr
    i4 = ux0v[r, :] + (ix1v[r, :] - ux0v[r, :]) * mfr
    ivec = [ilat[r, pl.ds(j * L, L)] for j in range(4)] + [i4]
    uvec = [ulat[r, pl.ds(j * L, L)] for j in range(4)] + [u4]
    mx = jnp.maximum(jnp.maximum(jnp.maximum(ivec[0], ivec[1]),
                                 jnp.maximum(ivec[2], ivec[3])), ivec[4])
    m_s = jnp.max(mx)
    e = [jnp.exp(v - m_s) for v in ivec]
    s = [1.0 / (1.0 + jnp.exp(-v)) for v in uvec]
    evec = (e[0] + e[1]) + (e[2] + e[3]) + e[4]
    pvec = (s[0] * e[0] + s[1] * e[1]) + (s[2] * e[2] + s[3] * e[3]) + s[4] * e[4]
    valvec = jnp.full((L,), jnp.sum(pvec), jnp.float32) / jnp.full(
        (L,), jnp.sum(evec), jnp.float32)
    plsc.store_scatter(outv, [rsplat], valvec, mask=lane == 0)

  del row_body
  pltpu.sync_copy(outv, out_hbm.at[pl.ds(base, RPW)])


@jax.jit
def kernel(users, items, xij, user_table, item_table, item_xij1_table,
           item_xij0_table, user_xij1_table, user_xij0_table):
  del item_xij0_table

  mesh = plsc.VectorSubcoreMesh(core_axis_name="c", subcore_axis_name="s")
  run = functools.partial(
      pl.kernel,
      out_type=jax.ShapeDtypeStruct((B,), jnp.float32),
      mesh=mesh,
      compiler_params=pltpu.CompilerParams(needs_layout_passes=False,
                                           use_tc_tiling_on_sc=False),
      scratch_types=[
          pltpu.VMEM((RPW,), jnp.int32),
          pltpu.VMEM((RPW,), jnp.int32),
          pltpu.VMEM((RPW,), jnp.int32),
          pltpu.VMEM((RPW, LAT), jnp.float32),
          pltpu.VMEM((RPW, LAT), jnp.float32),
          pltpu.VMEM((RPW, XD), jnp.float32),
          pltpu.VMEM((RPW, XD), jnp.float32),
          pltpu.VMEM((RPW, XD), jnp.float32),
          pltpu.VMEM((RPW,), jnp.float32),
          pltpu.SemaphoreType.DMA,
      ],
  )(_sc_body)
  return run(users.astype(jnp.int32), items.astype(jnp.int32),
             xij.astype(jnp.int32), user_table, item_table,
             item_xij1_table, user_xij0_table, user_xij1_table)
